# baseline (speedup 1.0000x reference)
import functools

import jax
import jax.numpy as jnp
import numpy as np
from jax import lax
from jax.experimental import pallas as pl
from jax.experimental.pallas import tpu as pltpu

B, S, M, D, H, NH = 4, 32, 50000, 128, 128, 8
HD = H // NH
THR = 0.7
TM = 1000
NM = M // TM
NS = NH * S

_F32 = jnp.float32


def _prologue_body(ni_ref, w_in_ref, b_in_ref, wq_ref, bq_ref, wk_ref, bk_ref,
                   wv_ref, bv_ref, wi1_ref, bi1_ref, wi2_ref, bi2_ref, wo_ref,
                   imp_ref, kk_ref, kbias_ref, vb_ref, scale_ref, exists_ref,
                   sel_ref, selimp_ref, wosum_ref):
    ni = ni_ref[...]
    ip = jnp.dot(ni, w_in_ref[...], preferred_element_type=_F32) + b_in_ref[...]
    h1 = jnp.maximum(jnp.dot(ip, wi1_ref[...], preferred_element_type=_F32)
                     + bi1_ref[...], 0.0)
    logit = jnp.dot(h1, wi2_ref[...], preferred_element_type=_F32) + bi2_ref[...]
    imp = jax.nn.sigmoid(logit)
    imp_ref[...] = imp
    kmat = jnp.dot(ip, wk_ref[...], preferred_element_type=_F32) + bk_ref[...]
    vmat = jnp.dot(ip, wv_ref[...], preferred_element_type=_F32) + bv_ref[...]
    wq_eff = jnp.dot(w_in_ref[...], wq_ref[...], preferred_element_type=_F32)
    bq_eff = jnp.dot(b_in_ref[...], wq_ref[...], preferred_element_type=_F32) \
        + bq_ref[...]

    t1 = (lax.broadcasted_iota(jnp.int32, (S, NS), 1) % S
          == lax.broadcasted_iota(jnp.int32, (S, NS), 0)).astype(_F32)
    hm = (lax.broadcasted_iota(jnp.int32, (H, NS), 0) // HD
          == lax.broadcasted_iota(jnp.int32, (H, NS), 1) // S).astype(_F32)
    t2 = (lax.broadcasted_iota(jnp.int32, (NS, S), 0) % S
          == lax.broadcasted_iota(jnp.int32, (NS, S), 1)).astype(_F32)
    m2 = (lax.broadcasted_iota(jnp.int32, (NS, H), 0) // S
          == lax.broadcasted_iota(jnp.int32, (NS, H), 1) // HD).astype(_F32)
    iota_s = lax.broadcasted_iota(jnp.int32, (S, 1), 0)
    inv_sqrt_hd = np.float32(1.0 / np.sqrt(HD))
    wosum_ref[...] = jnp.sum(wo_ref[...], axis=1, keepdims=True)

    for b in range(B):
        k_b = kmat[b * S:(b + 1) * S, :]
        v_b = vmat[b * S:(b + 1) * S, :]
        kt_tiled = lax.dot_general(k_b, t1, (((0,), (0,)), ((), ())),
                                   preferred_element_type=_F32)
        kblock = kt_tiled * hm * inv_sqrt_hd
        kk_ref[b] = jnp.dot(wq_eff, kblock, preferred_element_type=_F32)
        kbias_ref[b] = jnp.dot(bq_eff, kblock, preferred_element_type=_F32)
        vb_ref[b] = jnp.dot(t2, v_b, preferred_element_type=_F32) * m2

        imp_b = imp[b * S:(b + 1) * S, :]
        scale_ref[b] = jnp.mean(imp_b).reshape(1, 1)
        mask_b = imp_b > THR
        exists_ref[b] = jnp.max(jnp.where(mask_b, 1.0, 0.0)).reshape(1, 1)
        last_b = jnp.maximum(jnp.max(jnp.where(mask_b, iota_s, -1)), 0)
        smask = iota_s == last_b
        ni_b = ni[b * S:(b + 1) * S, :]
        sel_ref[b] = jnp.sum(jnp.where(smask, ni_b, 0.0), axis=0, keepdims=True)
        selimp_ref[b] = jnp.sum(jnp.where(smask, imp_b, 0.0)).reshape(1, 1)


def _main_body(mb_ref, kk_ref, kbias_ref, vb_ref, ones_ref, wo_ref, bo_ref,
               scale_ref, wosum_ref, upd_ref, wr_ref, pos_ref, maxv_ref):
    i = pl.program_id(1)
    mb = mb_ref[0]
    scores = jnp.dot(mb, kk_ref[0], preferred_element_type=_F32) + kbias_ref[0]
    e = jnp.exp(scores - jnp.max(scores, axis=1, keepdims=True))
    ctxu = jnp.dot(e, vb_ref[0], preferred_element_type=_F32)
    denom = jnp.dot(e, ones_ref[...], preferred_element_type=_F32)
    ctx = ctxu / denom
    ww = jnp.dot(ctx, wo_ref[...], preferred_element_type=_F32) \
        + bo_ref[...]
    wr_ref[0] = ww * scale_ref[0]
    upd_ref[0] = mb

    sm = jnp.dot(ctx, wosum_ref[...], preferred_element_type=_F32)
    tmax = jnp.max(sm)
    iota = lax.broadcasted_iota(jnp.int32, (TM, 1), 0)
    tidx = jnp.min(jnp.where(sm == tmax, iota, TM))
    tmax2 = tmax.reshape(1, 1)
    posv = (i * TM + tidx).reshape(1, 1)

    @pl.when(i == 0)
    def _():
        maxv_ref[0] = tmax2
        pos_ref[0] = posv

    @pl.when(i > 0)
    def _():
        better = tmax2 > maxv_ref[0]
        maxv_ref[0] = jnp.where(better, tmax2, maxv_ref[0])
        pos_ref[0] = jnp.where(better, posv, pos_ref[0])


def _scatter_body(upd_in_ref, mb_any_ref, pos_ref, kk_ref, kbias_ref, vb_ref,
                  ones_ref, wo_ref, bo_ref, sel_ref, selimp_ref, exists_ref,
                  wu1_ref, bu1_ref, wu2_ref, bu2_ref,
                  out_ref, old_scratch, new_scratch, sem):
    copies = []
    for b in range(B):
        c = pltpu.make_async_copy(mb_any_ref.at[b, pl.ds(pos_ref[b], 1)],
                                  old_scratch.at[pl.ds(b, 1)], sem)
        c.start()
        copies.append(c)
    for c in copies:
        c.wait()
    old = old_scratch[...]
    sel = sel_ref[:, 0, :]
    comb = jnp.concatenate([old, sel], axis=1)
    a1 = jnp.maximum(jnp.dot(comb, wu1_ref[...], preferred_element_type=_F32)
                     + bu1_ref[...], 0.0)
    upd = jnp.tanh(jnp.dot(a1, wu2_ref[...], preferred_element_type=_F32)
                   + bu2_ref[...])
    for b in range(B):
        scores = jnp.dot(old[b:b + 1], kk_ref[b],
                         preferred_element_type=_F32) + kbias_ref[b]
        e = jnp.exp(scores - jnp.max(scores, axis=1, keepdims=True))
        ctxu = jnp.dot(e, vb_ref[b], preferred_element_type=_F32)
        denom = jnp.dot(e, ones_ref[...], preferred_element_type=_F32)
        wwrow = jnp.dot(ctxu / denom, wo_ref[...],
                        preferred_element_type=_F32) + bo_ref[...]
        wwpos = wwrow * selimp_ref[b]
        newv = jnp.where(exists_ref[b] > 0.5,
                         old[b:b + 1] + upd[b:b + 1] * wwpos, old[b:b + 1])
        new_scratch[b:b + 1, :] = newv
    copies = []
    for b in range(B):
        c = pltpu.make_async_copy(new_scratch.at[pl.ds(b, 1)],
                                  out_ref.at[b, pl.ds(pos_ref[b], 1)], sem)
        c.start()
        copies.append(c)
    for c in copies:
        c.wait()


def kernel(new_info, memory_bank, W_in, b_in, Wq, bq, Wk, bk, Wv, bv, Wo, bo,
           W_imp1, b_imp1, W_imp2, b_imp2, W_upd1, b_upd1, W_upd2, b_upd2):
    ni2 = new_info.reshape(B * S, D)
    row2 = lambda x: x.reshape(1, -1)

    imp2, kk, kbias, vb, scale, exists, sel, selimp, wosum = pl.pallas_call(
        _prologue_body,
        out_shape=(
            jax.ShapeDtypeStruct((B * S, 1), _F32),
            jax.ShapeDtypeStruct((B, D, NS), _F32),
            jax.ShapeDtypeStruct((B, 1, NS), _F32),
            jax.ShapeDtypeStruct((B, NS, H), _F32),
            jax.ShapeDtypeStruct((B, 1, 1), _F32),
            jax.ShapeDtypeStruct((B, 1, 1), _F32),
            jax.ShapeDtypeStruct((B, 1, D), _F32),
            jax.ShapeDtypeStruct((B, 1, 1), _F32),
            jax.ShapeDtypeStruct((D, 1), _F32),
        ),
    )(ni2, W_in, row2(b_in), Wq, row2(bq), Wk, row2(bk), Wv, row2(bv),
      W_imp1, row2(b_imp1), W_imp2, row2(b_imp2), Wo)

    ones16 = jnp.asarray(
        np.kron(np.eye(NH, dtype=np.float32), np.ones((S, HD), np.float32)))

    grid = (B, NM)
    upd_copy, ww_ret, pos, _maxv = pl.pallas_call(
        _main_body,
        grid=grid,
        in_specs=[
            pl.BlockSpec((1, TM, D), lambda b, i: (b, i, 0)),
            pl.BlockSpec((1, D, NS), lambda b, i: (b, 0, 0)),
            pl.BlockSpec((1, 1, NS), lambda b, i: (b, 0, 0)),
            pl.BlockSpec((1, NS, H), lambda b, i: (b, 0, 0)),
            pl.BlockSpec((NS, H), lambda b, i: (0, 0)),
            pl.BlockSpec((D, H), lambda b, i: (0, 0)),
            pl.BlockSpec((1, H), lambda b, i: (0, 0)),
            pl.BlockSpec((1, 1, 1), lambda b, i: (b, 0, 0)),
            pl.BlockSpec((D, 1), lambda b, i: (0, 0)),
        ],
        out_specs=[
            pl.BlockSpec((1, TM, D), lambda b, i: (b, i, 0)),
            pl.BlockSpec((1, TM, H), lambda b, i: (b, i, 0)),
            pl.BlockSpec((1, 1, 1), lambda b, i: (b, 0, 0)),
            pl.BlockSpec((1, 1, 1), lambda b, i: (b, 0, 0)),
        ],
        out_shape=(
            jax.ShapeDtypeStruct((B, M, D), _F32),
            jax.ShapeDtypeStruct((B, M, H), _F32),
            jax.ShapeDtypeStruct((B, 1, 1), jnp.int32),
            jax.ShapeDtypeStruct((B, 1, 1), _F32),
        ),
    )(memory_bank, kk, kbias, vb, ones16, Wo, row2(bo), scale, wosum)

    vspec = pl.BlockSpec(memory_space=pltpu.VMEM)
    updated = pl.pallas_call(
        _scatter_body,
        in_specs=[
            pl.BlockSpec(memory_space=pl.ANY),
            pl.BlockSpec(memory_space=pl.ANY),
            pl.BlockSpec(memory_space=pltpu.SMEM),
            vspec, vspec, vspec, vspec, vspec, vspec, vspec, vspec, vspec,
            vspec, vspec, vspec, vspec,
        ],
        out_specs=pl.BlockSpec(memory_space=pl.ANY),
        out_shape=jax.ShapeDtypeStruct((B, M, D), _F32),
        input_output_aliases={0: 0},
        scratch_shapes=[pltpu.VMEM((B, D), _F32), pltpu.VMEM((B, D), _F32),
                        pltpu.SemaphoreType.DMA],
    )(upd_copy, memory_bank, pos.reshape(B), kk, kbias, vb, ones16, Wo,
      row2(bo), sel, selimp, exists, W_upd1, row2(b_upd1), W_upd2,
      row2(b_upd2))

    return updated, ww_ret, imp2.reshape(B, S)


# speedup vs baseline: 1.1261x; 1.1261x over previous
import functools

import jax
import jax.numpy as jnp
import numpy as np
from jax import lax
from jax.experimental import pallas as pl
from jax.experimental.pallas import tpu as pltpu

B, S, M, D, H, NH = 4, 32, 50000, 128, 128, 8
HD = H // NH
THR = 0.7
TM = 1000
NM = M // TM
NS = NH * S

_F32 = jnp.float32


def _prologue_body(ni_ref, w_in_ref, b_in_ref, wq_ref, bq_ref, wk_ref, bk_ref,
                   wv_ref, bv_ref, wi1_ref, bi1_ref, wi2_ref, bi2_ref, wo_ref,
                   imp_ref, kk_ref, kbias_ref, vb_ref, scale_ref, exists_ref,
                   sel_ref, selimp_ref):
    ni = ni_ref[...]
    ip = jnp.dot(ni, w_in_ref[...], preferred_element_type=_F32) + b_in_ref[...]
    h1 = jnp.maximum(jnp.dot(ip, wi1_ref[...], preferred_element_type=_F32)
                     + bi1_ref[...], 0.0)
    logit = jnp.dot(h1, wi2_ref[...], preferred_element_type=_F32) + bi2_ref[...]
    imp = jax.nn.sigmoid(logit)
    imp_ref[...] = imp
    kmat = jnp.dot(ip, wk_ref[...], preferred_element_type=_F32) + bk_ref[...]
    vmat = jnp.dot(ip, wv_ref[...], preferred_element_type=_F32) + bv_ref[...]
    wq_eff = jnp.dot(w_in_ref[...], wq_ref[...], preferred_element_type=_F32)
    bq_eff = jnp.dot(b_in_ref[...], wq_ref[...], preferred_element_type=_F32) \
        + bq_ref[...]

    t1 = (lax.broadcasted_iota(jnp.int32, (S, NS), 1) % S
          == lax.broadcasted_iota(jnp.int32, (S, NS), 0)).astype(_F32)
    hm = (lax.broadcasted_iota(jnp.int32, (H, NS), 0) // HD
          == lax.broadcasted_iota(jnp.int32, (H, NS), 1) // S).astype(_F32)
    t2 = (lax.broadcasted_iota(jnp.int32, (NS, S), 0) % S
          == lax.broadcasted_iota(jnp.int32, (NS, S), 1)).astype(_F32)
    m2 = (lax.broadcasted_iota(jnp.int32, (NS, H), 0) // S
          == lax.broadcasted_iota(jnp.int32, (NS, H), 1) // HD).astype(_F32)
    iota_s = lax.broadcasted_iota(jnp.int32, (S, 1), 0)
    inv_sqrt_hd = np.float32(1.0 / np.sqrt(HD))

    for b in range(B):
        k_b = kmat[b * S:(b + 1) * S, :]
        v_b = vmat[b * S:(b + 1) * S, :]
        kt_tiled = lax.dot_general(k_b, t1, (((0,), (0,)), ((), ())),
                                   preferred_element_type=_F32)
        kblock = kt_tiled * hm * inv_sqrt_hd
        kk_ref[b] = jnp.dot(wq_eff, kblock, preferred_element_type=_F32)
        kbias_ref[b] = jnp.dot(bq_eff, kblock, preferred_element_type=_F32)
        vb_ref[b] = jnp.dot(t2, v_b, preferred_element_type=_F32) * m2

        imp_b = imp[b * S:(b + 1) * S, :]
        scale_ref[b] = jnp.mean(imp_b).reshape(1, 1)
        mask_b = imp_b > THR
        exists_ref[b] = jnp.max(jnp.where(mask_b, 1.0, 0.0)).reshape(1, 1)
        last_b = jnp.maximum(jnp.max(jnp.where(mask_b, iota_s, -1)), 0)
        smask = iota_s == last_b
        ni_b = ni[b * S:(b + 1) * S, :]
        sel_ref[b] = jnp.sum(jnp.where(smask, ni_b, 0.0), axis=0, keepdims=True)
        selimp_ref[b] = jnp.sum(jnp.where(smask, imp_b, 0.0)).reshape(1, 1)


def _main_body(mb_ref, kk_ref, kbias_ref, vb_ref, ones_ref, wo_ref, bo_ref,
               scale_ref, upd_ref, wr_ref, pos_ref, maxv_ref):
    i = pl.program_id(1)
    mb = mb_ref[0]
    scores = jnp.dot(mb, kk_ref[0], preferred_element_type=_F32) + kbias_ref[0]
    e = jnp.exp(scores - jnp.max(scores, axis=1, keepdims=True))
    ctxu = jnp.dot(e, vb_ref[0], preferred_element_type=_F32)
    denom = jnp.dot(e, ones_ref[...], preferred_element_type=_F32)
    ctx = ctxu / denom
    ww = jnp.dot(ctx, wo_ref[...], preferred_element_type=_F32) \
        + bo_ref[...]
    wr_ref[0] = ww * scale_ref[0]
    upd_ref[0] = mb

    sm = jnp.mean(ww, axis=1, keepdims=True)
    tmax = jnp.max(sm)
    iota = lax.broadcasted_iota(jnp.int32, (TM, 1), 0)
    tidx = jnp.min(jnp.where(sm == tmax, iota, TM))
    tmax2 = tmax.reshape(1, 1)
    posv = (i * TM + tidx).reshape(1, 1)

    @pl.when(i == 0)
    def _():
        maxv_ref[0] = tmax2
        pos_ref[0] = posv

    @pl.when(i > 0)
    def _():
        better = tmax2 > maxv_ref[0]
        maxv_ref[0] = jnp.where(better, tmax2, maxv_ref[0])
        pos_ref[0] = jnp.where(better, posv, pos_ref[0])


def _scatter_body(upd_in_ref, mb_any_ref, pos_ref, kk_ref, kbias_ref, vb_ref,
                  ones_ref, wo_ref, bo_ref, sel_ref, selimp_ref, exists_ref,
                  wu1_ref, bu1_ref, wu2_ref, bu2_ref,
                  out_ref, old_scratch, new_scratch, sem):
    copies = []
    for b in range(B):
        c = pltpu.make_async_copy(mb_any_ref.at[b, pl.ds(pos_ref[b], 1)],
                                  old_scratch.at[pl.ds(b, 1)], sem)
        c.start()
        copies.append(c)
    for c in copies:
        c.wait()
    old = old_scratch[...]
    sel = sel_ref[:, 0, :]
    comb = jnp.concatenate([old, sel], axis=1)
    a1 = jnp.maximum(jnp.dot(comb, wu1_ref[...], preferred_element_type=_F32)
                     + bu1_ref[...], 0.0)
    upd = jnp.tanh(jnp.dot(a1, wu2_ref[...], preferred_element_type=_F32)
                   + bu2_ref[...])
    for b in range(B):
        scores = jnp.dot(old[b:b + 1], kk_ref[b],
                         preferred_element_type=_F32) + kbias_ref[b]
        e = jnp.exp(scores - jnp.max(scores, axis=1, keepdims=True))
        ctxu = jnp.dot(e, vb_ref[b], preferred_element_type=_F32)
        denom = jnp.dot(e, ones_ref[...], preferred_element_type=_F32)
        wwrow = jnp.dot(ctxu / denom, wo_ref[...],
                        preferred_element_type=_F32) + bo_ref[...]
        wwpos = wwrow * selimp_ref[b]
        newv = jnp.where(exists_ref[b] > 0.5,
                         old[b:b + 1] + upd[b:b + 1] * wwpos, old[b:b + 1])
        new_scratch[b:b + 1, :] = newv
    copies = []
    for b in range(B):
        c = pltpu.make_async_copy(new_scratch.at[pl.ds(b, 1)],
                                  out_ref.at[b, pl.ds(pos_ref[b], 1)], sem)
        c.start()
        copies.append(c)
    for c in copies:
        c.wait()


def kernel(new_info, memory_bank, W_in, b_in, Wq, bq, Wk, bk, Wv, bv, Wo, bo,
           W_imp1, b_imp1, W_imp2, b_imp2, W_upd1, b_upd1, W_upd2, b_upd2):
    ni2 = new_info.reshape(B * S, D)
    row2 = lambda x: x.reshape(1, -1)

    imp2, kk, kbias, vb, scale, exists, sel, selimp = pl.pallas_call(
        _prologue_body,
        out_shape=(
            jax.ShapeDtypeStruct((B * S, 1), _F32),
            jax.ShapeDtypeStruct((B, D, NS), _F32),
            jax.ShapeDtypeStruct((B, 1, NS), _F32),
            jax.ShapeDtypeStruct((B, NS, H), _F32),
            jax.ShapeDtypeStruct((B, 1, 1), _F32),
            jax.ShapeDtypeStruct((B, 1, 1), _F32),
            jax.ShapeDtypeStruct((B, 1, D), _F32),
            jax.ShapeDtypeStruct((B, 1, 1), _F32),
        ),
    )(ni2, W_in, row2(b_in), Wq, row2(bq), Wk, row2(bk), Wv, row2(bv),
      W_imp1, row2(b_imp1), W_imp2, row2(b_imp2), Wo)

    ones16 = jnp.asarray(
        np.kron(np.eye(NH, dtype=np.float32), np.ones((S, HD), np.float32)))

    grid = (B, NM)
    upd_copy, ww_ret, pos, _maxv = pl.pallas_call(
        _main_body,
        grid=grid,
        in_specs=[
            pl.BlockSpec((1, TM, D), lambda b, i: (b, i, 0)),
            pl.BlockSpec((1, D, NS), lambda b, i: (b, 0, 0)),
            pl.BlockSpec((1, 1, NS), lambda b, i: (b, 0, 0)),
            pl.BlockSpec((1, NS, H), lambda b, i: (b, 0, 0)),
            pl.BlockSpec((NS, H), lambda b, i: (0, 0)),
            pl.BlockSpec((D, H), lambda b, i: (0, 0)),
            pl.BlockSpec((1, H), lambda b, i: (0, 0)),
            pl.BlockSpec((1, 1, 1), lambda b, i: (b, 0, 0)),
        ],
        out_specs=[
            pl.BlockSpec((1, TM, D), lambda b, i: (b, i, 0)),
            pl.BlockSpec((1, TM, H), lambda b, i: (b, i, 0)),
            pl.BlockSpec((1, 1, 1), lambda b, i: (b, 0, 0)),
            pl.BlockSpec((1, 1, 1), lambda b, i: (b, 0, 0)),
        ],
        out_shape=(
            jax.ShapeDtypeStruct((B, M, D), _F32),
            jax.ShapeDtypeStruct((B, M, H), _F32),
            jax.ShapeDtypeStruct((B, 1, 1), jnp.int32),
            jax.ShapeDtypeStruct((B, 1, 1), _F32),
        ),
    )(memory_bank, kk, kbias, vb, ones16, Wo, row2(bo), scale)

    vspec = pl.BlockSpec(memory_space=pltpu.VMEM)
    updated = pl.pallas_call(
        _scatter_body,
        in_specs=[
            pl.BlockSpec(memory_space=pl.ANY),
            pl.BlockSpec(memory_space=pl.ANY),
            pl.BlockSpec(memory_space=pltpu.SMEM),
            vspec, vspec, vspec, vspec, vspec, vspec, vspec, vspec, vspec,
            vspec, vspec, vspec, vspec,
        ],
        out_specs=pl.BlockSpec(memory_space=pl.ANY),
        out_shape=jax.ShapeDtypeStruct((B, M, D), _F32),
        input_output_aliases={0: 0},
        scratch_shapes=[pltpu.VMEM((B, D), _F32), pltpu.VMEM((B, D), _F32),
                        pltpu.SemaphoreType.DMA],
    )(upd_copy, memory_bank, pos.reshape(B), kk, kbias, vb, ones16, Wo,
      row2(bo), sel, selimp, exists, W_upd1, row2(b_upd1), W_upd2,
      row2(b_upd2))

    return updated, ww_ret, imp2.reshape(B, S)


# speedup vs baseline: 1.5701x; 1.3942x over previous
import functools

import jax
import jax.numpy as jnp
import numpy as np
from jax import lax
from jax.experimental import pallas as pl
from jax.experimental.pallas import tpu as pltpu

B, S, M, D, H, NH = 4, 32, 50000, 128, 128, 8
HD = H // NH
THR = 0.7
TM = 2000
NM = M // TM
NS = NH * S

_F32 = jnp.float32


def _prologue_body(ni_ref, w_in_ref, b_in_ref, wq_ref, bq_ref, wk_ref, bk_ref,
                   wv_ref, bv_ref, wi1_ref, bi1_ref, wi2_ref, bi2_ref, wo_ref,
                   imp_ref, kk_ref, kbias_ref, vb_ref, scale_ref, exists_ref,
                   sel_ref, selimp_ref):
    ni = ni_ref[...]
    ip = jnp.dot(ni, w_in_ref[...], preferred_element_type=_F32) + b_in_ref[...]
    h1 = jnp.maximum(jnp.dot(ip, wi1_ref[...], preferred_element_type=_F32)
                     + bi1_ref[...], 0.0)
    logit = jnp.dot(h1, wi2_ref[...], preferred_element_type=_F32) + bi2_ref[...]
    imp = jax.nn.sigmoid(logit)
    imp_ref[...] = imp
    kmat = jnp.dot(ip, wk_ref[...], preferred_element_type=_F32) + bk_ref[...]
    vmat = jnp.dot(ip, wv_ref[...], preferred_element_type=_F32) + bv_ref[...]
    wq_eff = jnp.dot(w_in_ref[...], wq_ref[...], preferred_element_type=_F32)
    bq_eff = jnp.dot(b_in_ref[...], wq_ref[...], preferred_element_type=_F32) \
        + bq_ref[...]

    t1 = (lax.broadcasted_iota(jnp.int32, (S, NS), 1) % S
          == lax.broadcasted_iota(jnp.int32, (S, NS), 0)).astype(_F32)
    hm = (lax.broadcasted_iota(jnp.int32, (H, NS), 0) // HD
          == lax.broadcasted_iota(jnp.int32, (H, NS), 1) // S).astype(_F32)
    t2 = (lax.broadcasted_iota(jnp.int32, (NS, S), 0) % S
          == lax.broadcasted_iota(jnp.int32, (NS, S), 1)).astype(_F32)
    m2 = (lax.broadcasted_iota(jnp.int32, (NS, H), 0) // S
          == lax.broadcasted_iota(jnp.int32, (NS, H), 1) // HD).astype(_F32)
    iota_s = lax.broadcasted_iota(jnp.int32, (S, 1), 0)
    inv_sqrt_hd = np.float32(1.0 / np.sqrt(HD))

    for b in range(B):
        k_b = kmat[b * S:(b + 1) * S, :]
        v_b = vmat[b * S:(b + 1) * S, :]
        kt_tiled = lax.dot_general(k_b, t1, (((0,), (0,)), ((), ())),
                                   preferred_element_type=_F32)
        kblock = kt_tiled * hm * inv_sqrt_hd
        kk_ref[b] = jnp.dot(wq_eff, kblock, preferred_element_type=_F32)
        kbias_ref[b] = jnp.dot(bq_eff, kblock, preferred_element_type=_F32)
        vb_ref[b] = jnp.dot(t2, v_b, preferred_element_type=_F32) * m2

        imp_b = imp[b * S:(b + 1) * S, :]
        scale_ref[b] = jnp.mean(imp_b).reshape(1, 1)
        mask_b = imp_b > THR
        exists_ref[b] = jnp.max(jnp.where(mask_b, 1.0, 0.0)).reshape(1, 1)
        last_b = jnp.maximum(jnp.max(jnp.where(mask_b, iota_s, -1)), 0)
        smask = iota_s == last_b
        ni_b = ni[b * S:(b + 1) * S, :]
        sel_ref[b] = jnp.sum(jnp.where(smask, ni_b, 0.0), axis=0, keepdims=True)
        selimp_ref[b] = jnp.sum(jnp.where(smask, imp_b, 0.0)).reshape(1, 1)


def _main_body(mb_ref, kk_ref, kbias_ref, vb_ref, ones_ref, wo_ref, bo_ref,
               scale_ref, upd_ref, wr_ref, pos_ref, maxv_ref):
    i = pl.program_id(1)
    mb = mb_ref[0]
    scores = jnp.dot(mb, kk_ref[0], preferred_element_type=_F32) + kbias_ref[0]
    e = jnp.exp(scores - jnp.max(scores, axis=1, keepdims=True))
    ctxu = jnp.dot(e, vb_ref[0], preferred_element_type=_F32)
    denom = jnp.dot(e, ones_ref[...], preferred_element_type=_F32)
    ctx = ctxu / denom
    ww = jnp.dot(ctx, wo_ref[...], preferred_element_type=_F32) \
        + bo_ref[...]
    wr_ref[0] = ww * scale_ref[0]
    upd_ref[0] = mb

    sm = jnp.mean(ww, axis=1, keepdims=True)
    tmax = jnp.max(sm)
    iota = lax.broadcasted_iota(jnp.int32, (TM, 1), 0)
    tidx = jnp.min(jnp.where(sm == tmax, iota, TM))
    tmax2 = tmax.reshape(1, 1)
    posv = (i * TM + tidx).reshape(1, 1)

    @pl.when(i == 0)
    def _():
        maxv_ref[0] = tmax2
        pos_ref[0] = posv

    @pl.when(i > 0)
    def _():
        better = tmax2 > maxv_ref[0]
        maxv_ref[0] = jnp.where(better, tmax2, maxv_ref[0])
        pos_ref[0] = jnp.where(better, posv, pos_ref[0])


def _scatter_body(upd_in_ref, mb_any_ref, pos_ref, kk_ref, kbias_ref, vb_ref,
                  ones_ref, wo_ref, bo_ref, sel_ref, selimp_ref, exists_ref,
                  wu1_ref, bu1_ref, wu2_ref, bu2_ref,
                  out_ref, old_scratch, new_scratch, sem):
    copies = []
    for b in range(B):
        c = pltpu.make_async_copy(mb_any_ref.at[b, pl.ds(pos_ref[b], 1)],
                                  old_scratch.at[pl.ds(b, 1)], sem)
        c.start()
        copies.append(c)
    for c in copies:
        c.wait()
    old = old_scratch[...]
    sel = sel_ref[:, 0, :]
    comb = jnp.concatenate([old, sel], axis=1)
    a1 = jnp.maximum(jnp.dot(comb, wu1_ref[...], preferred_element_type=_F32)
                     + bu1_ref[...], 0.0)
    upd = jnp.tanh(jnp.dot(a1, wu2_ref[...], preferred_element_type=_F32)
                   + bu2_ref[...])
    for b in range(B):
        scores = jnp.dot(old[b:b + 1], kk_ref[b],
                         preferred_element_type=_F32) + kbias_ref[b]
        e = jnp.exp(scores - jnp.max(scores, axis=1, keepdims=True))
        ctxu = jnp.dot(e, vb_ref[b], preferred_element_type=_F32)
        denom = jnp.dot(e, ones_ref[...], preferred_element_type=_F32)
        wwrow = jnp.dot(ctxu / denom, wo_ref[...],
                        preferred_element_type=_F32) + bo_ref[...]
        wwpos = wwrow * selimp_ref[b]
        newv = jnp.where(exists_ref[b] > 0.5,
                         old[b:b + 1] + upd[b:b + 1] * wwpos, old[b:b + 1])
        new_scratch[b:b + 1, :] = newv
    copies = []
    for b in range(B):
        c = pltpu.make_async_copy(new_scratch.at[pl.ds(b, 1)],
                                  out_ref.at[b, pl.ds(pos_ref[b], 1)], sem)
        c.start()
        copies.append(c)
    for c in copies:
        c.wait()


def kernel(new_info, memory_bank, W_in, b_in, Wq, bq, Wk, bk, Wv, bv, Wo, bo,
           W_imp1, b_imp1, W_imp2, b_imp2, W_upd1, b_upd1, W_upd2, b_upd2):
    ni2 = new_info.reshape(B * S, D)
    row2 = lambda x: x.reshape(1, -1)

    imp2, kk, kbias, vb, scale, exists, sel, selimp = pl.pallas_call(
        _prologue_body,
        out_shape=(
            jax.ShapeDtypeStruct((B * S, 1), _F32),
            jax.ShapeDtypeStruct((B, D, NS), _F32),
            jax.ShapeDtypeStruct((B, 1, NS), _F32),
            jax.ShapeDtypeStruct((B, NS, H), _F32),
            jax.ShapeDtypeStruct((B, 1, 1), _F32),
            jax.ShapeDtypeStruct((B, 1, 1), _F32),
            jax.ShapeDtypeStruct((B, 1, D), _F32),
            jax.ShapeDtypeStruct((B, 1, 1), _F32),
        ),
    )(ni2, W_in, row2(b_in), Wq, row2(bq), Wk, row2(bk), Wv, row2(bv),
      W_imp1, row2(b_imp1), W_imp2, row2(b_imp2), Wo)

    ones16 = jnp.asarray(
        np.kron(np.eye(NH, dtype=np.float32), np.ones((S, HD), np.float32)))

    grid = (B, NM)
    upd_copy, ww_ret, pos, _maxv = pl.pallas_call(
        _main_body,
        grid=grid,
        in_specs=[
            pl.BlockSpec((1, TM, D), lambda b, i: (b, i, 0)),
            pl.BlockSpec((1, D, NS), lambda b, i: (b, 0, 0)),
            pl.BlockSpec((1, 1, NS), lambda b, i: (b, 0, 0)),
            pl.BlockSpec((1, NS, H), lambda b, i: (b, 0, 0)),
            pl.BlockSpec((NS, H), lambda b, i: (0, 0)),
            pl.BlockSpec((D, H), lambda b, i: (0, 0)),
            pl.BlockSpec((1, H), lambda b, i: (0, 0)),
            pl.BlockSpec((1, 1, 1), lambda b, i: (b, 0, 0)),
        ],
        out_specs=[
            pl.BlockSpec((1, TM, D), lambda b, i: (b, i, 0)),
            pl.BlockSpec((1, TM, H), lambda b, i: (b, i, 0)),
            pl.BlockSpec((1, 1, 1), lambda b, i: (b, 0, 0)),
            pl.BlockSpec((1, 1, 1), lambda b, i: (b, 0, 0)),
        ],
        out_shape=(
            jax.ShapeDtypeStruct((B, M, D), _F32),
            jax.ShapeDtypeStruct((B, M, H), _F32),
            jax.ShapeDtypeStruct((B, 1, 1), jnp.int32),
            jax.ShapeDtypeStruct((B, 1, 1), _F32),
        ),
    )(memory_bank, kk, kbias, vb, ones16, Wo, row2(bo), scale)

    vspec = pl.BlockSpec(memory_space=pltpu.VMEM)
    updated = pl.pallas_call(
        _scatter_body,
        in_specs=[
            pl.BlockSpec(memory_space=pl.ANY),
            pl.BlockSpec(memory_space=pl.ANY),
            pl.BlockSpec(memory_space=pltpu.SMEM),
            vspec, vspec, vspec, vspec, vspec, vspec, vspec, vspec, vspec,
            vspec, vspec, vspec, vspec,
        ],
        out_specs=pl.BlockSpec(memory_space=pl.ANY),
        out_shape=jax.ShapeDtypeStruct((B, M, D), _F32),
        input_output_aliases={0: 0},
        scratch_shapes=[pltpu.VMEM((B, D), _F32), pltpu.VMEM((B, D), _F32),
                        pltpu.SemaphoreType.DMA],
    )(upd_copy, memory_bank, pos.reshape(B), kk, kbias, vb, ones16, Wo,
      row2(bo), sel, selimp, exists, W_upd1, row2(b_upd1), W_upd2,
      row2(b_upd2))

    return updated, ww_ret, imp2.reshape(B, S)


# speedup vs baseline: 1.8404x; 1.1722x over previous
import functools

import jax
import jax.numpy as jnp
import numpy as np
from jax import lax
from jax.experimental import pallas as pl
from jax.experimental.pallas import tpu as pltpu

B, S, M, D, H, NH = 4, 32, 50000, 128, 128, 8
HD = H // NH
THR = 0.7
TM = 5000
NM = M // TM
NS = NH * S

_F32 = jnp.float32


def _prologue_body(ni_ref, w_in_ref, b_in_ref, wq_ref, bq_ref, wk_ref, bk_ref,
                   wv_ref, bv_ref, wi1_ref, bi1_ref, wi2_ref, bi2_ref, wo_ref,
                   imp_ref, kk_ref, kbias_ref, vb_ref, scale_ref, exists_ref,
                   sel_ref, selimp_ref):
    ni = ni_ref[...]
    ip = jnp.dot(ni, w_in_ref[...], preferred_element_type=_F32) + b_in_ref[...]
    h1 = jnp.maximum(jnp.dot(ip, wi1_ref[...], preferred_element_type=_F32)
                     + bi1_ref[...], 0.0)
    logit = jnp.dot(h1, wi2_ref[...], preferred_element_type=_F32) + bi2_ref[...]
    imp = jax.nn.sigmoid(logit)
    imp_ref[...] = imp
    kmat = jnp.dot(ip, wk_ref[...], preferred_element_type=_F32) + bk_ref[...]
    vmat = jnp.dot(ip, wv_ref[...], preferred_element_type=_F32) + bv_ref[...]
    wq_eff = jnp.dot(w_in_ref[...], wq_ref[...], preferred_element_type=_F32)
    bq_eff = jnp.dot(b_in_ref[...], wq_ref[...], preferred_element_type=_F32) \
        + bq_ref[...]

    t1 = (lax.broadcasted_iota(jnp.int32, (S, NS), 1) % S
          == lax.broadcasted_iota(jnp.int32, (S, NS), 0)).astype(_F32)
    hm = (lax.broadcasted_iota(jnp.int32, (H, NS), 0) // HD
          == lax.broadcasted_iota(jnp.int32, (H, NS), 1) // S).astype(_F32)
    t2 = (lax.broadcasted_iota(jnp.int32, (NS, S), 0) % S
          == lax.broadcasted_iota(jnp.int32, (NS, S), 1)).astype(_F32)
    m2 = (lax.broadcasted_iota(jnp.int32, (NS, H), 0) // S
          == lax.broadcasted_iota(jnp.int32, (NS, H), 1) // HD).astype(_F32)
    iota_s = lax.broadcasted_iota(jnp.int32, (S, 1), 0)
    inv_sqrt_hd = np.float32(1.0 / np.sqrt(HD))

    for b in range(B):
        k_b = kmat[b * S:(b + 1) * S, :]
        v_b = vmat[b * S:(b + 1) * S, :]
        kt_tiled = lax.dot_general(k_b, t1, (((0,), (0,)), ((), ())),
                                   preferred_element_type=_F32)
        kblock = kt_tiled * hm * inv_sqrt_hd
        kk_ref[b] = jnp.dot(wq_eff, kblock, preferred_element_type=_F32)
        kbias_ref[b] = jnp.dot(bq_eff, kblock, preferred_element_type=_F32)
        vb_ref[b] = jnp.dot(t2, v_b, preferred_element_type=_F32) * m2

        imp_b = imp[b * S:(b + 1) * S, :]
        scale_ref[b] = jnp.mean(imp_b).reshape(1, 1)
        mask_b = imp_b > THR
        exists_ref[b] = jnp.max(jnp.where(mask_b, 1.0, 0.0)).reshape(1, 1)
        last_b = jnp.maximum(jnp.max(jnp.where(mask_b, iota_s, -1)), 0)
        smask = iota_s == last_b
        ni_b = ni[b * S:(b + 1) * S, :]
        sel_ref[b] = jnp.sum(jnp.where(smask, ni_b, 0.0), axis=0, keepdims=True)
        selimp_ref[b] = jnp.sum(jnp.where(smask, imp_b, 0.0)).reshape(1, 1)


def _main_body(mb_ref, kk_ref, kbias_ref, vb_ref, ones_ref, wo_ref, bo_ref,
               scale_ref, upd_ref, wr_ref, pos_ref, maxv_ref):
    i = pl.program_id(1)
    mb = mb_ref[0]
    scores = jnp.dot(mb, kk_ref[0], preferred_element_type=_F32) + kbias_ref[0]
    e = jnp.exp(scores - jnp.max(scores, axis=1, keepdims=True))
    ctxu = jnp.dot(e, vb_ref[0], preferred_element_type=_F32)
    denom = jnp.dot(e, ones_ref[...], preferred_element_type=_F32)
    ctx = ctxu / denom
    ww = jnp.dot(ctx, wo_ref[...], preferred_element_type=_F32) \
        + bo_ref[...]
    wr_ref[0] = ww * scale_ref[0]
    upd_ref[0] = mb

    sm = jnp.mean(ww, axis=1, keepdims=True)
    tmax = jnp.max(sm)
    iota = lax.broadcasted_iota(jnp.int32, (TM, 1), 0)
    tidx = jnp.min(jnp.where(sm == tmax, iota, TM))
    tmax2 = tmax.reshape(1, 1)
    posv = (i * TM + tidx).reshape(1, 1)

    @pl.when(i == 0)
    def _():
        maxv_ref[0] = tmax2
        pos_ref[0] = posv

    @pl.when(i > 0)
    def _():
        better = tmax2 > maxv_ref[0]
        maxv_ref[0] = jnp.where(better, tmax2, maxv_ref[0])
        pos_ref[0] = jnp.where(better, posv, pos_ref[0])


def _scatter_body(upd_in_ref, mb_any_ref, pos_ref, kk_ref, kbias_ref, vb_ref,
                  ones_ref, wo_ref, bo_ref, sel_ref, selimp_ref, exists_ref,
                  wu1_ref, bu1_ref, wu2_ref, bu2_ref,
                  out_ref, old_scratch, new_scratch, sem):
    copies = []
    for b in range(B):
        c = pltpu.make_async_copy(mb_any_ref.at[b, pl.ds(pos_ref[b], 1)],
                                  old_scratch.at[pl.ds(b, 1)], sem)
        c.start()
        copies.append(c)
    for c in copies:
        c.wait()
    old = old_scratch[...]
    sel = sel_ref[:, 0, :]
    comb = jnp.concatenate([old, sel], axis=1)
    a1 = jnp.maximum(jnp.dot(comb, wu1_ref[...], preferred_element_type=_F32)
                     + bu1_ref[...], 0.0)
    upd = jnp.tanh(jnp.dot(a1, wu2_ref[...], preferred_element_type=_F32)
                   + bu2_ref[...])
    for b in range(B):
        scores = jnp.dot(old[b:b + 1], kk_ref[b],
                         preferred_element_type=_F32) + kbias_ref[b]
        e = jnp.exp(scores - jnp.max(scores, axis=1, keepdims=True))
        ctxu = jnp.dot(e, vb_ref[b], preferred_element_type=_F32)
        denom = jnp.dot(e, ones_ref[...], preferred_element_type=_F32)
        wwrow = jnp.dot(ctxu / denom, wo_ref[...],
                        preferred_element_type=_F32) + bo_ref[...]
        wwpos = wwrow * selimp_ref[b]
        newv = jnp.where(exists_ref[b] > 0.5,
                         old[b:b + 1] + upd[b:b + 1] * wwpos, old[b:b + 1])
        new_scratch[b:b + 1, :] = newv
    copies = []
    for b in range(B):
        c = pltpu.make_async_copy(new_scratch.at[pl.ds(b, 1)],
                                  out_ref.at[b, pl.ds(pos_ref[b], 1)], sem)
        c.start()
        copies.append(c)
    for c in copies:
        c.wait()


def kernel(new_info, memory_bank, W_in, b_in, Wq, bq, Wk, bk, Wv, bv, Wo, bo,
           W_imp1, b_imp1, W_imp2, b_imp2, W_upd1, b_upd1, W_upd2, b_upd2):
    ni2 = new_info.reshape(B * S, D)
    row2 = lambda x: x.reshape(1, -1)

    imp2, kk, kbias, vb, scale, exists, sel, selimp = pl.pallas_call(
        _prologue_body,
        out_shape=(
            jax.ShapeDtypeStruct((B * S, 1), _F32),
            jax.ShapeDtypeStruct((B, D, NS), _F32),
            jax.ShapeDtypeStruct((B, 1, NS), _F32),
            jax.ShapeDtypeStruct((B, NS, H), _F32),
            jax.ShapeDtypeStruct((B, 1, 1), _F32),
            jax.ShapeDtypeStruct((B, 1, 1), _F32),
            jax.ShapeDtypeStruct((B, 1, D), _F32),
            jax.ShapeDtypeStruct((B, 1, 1), _F32),
        ),
    )(ni2, W_in, row2(b_in), Wq, row2(bq), Wk, row2(bk), Wv, row2(bv),
      W_imp1, row2(b_imp1), W_imp2, row2(b_imp2), Wo)

    ones16 = jnp.asarray(
        np.kron(np.eye(NH, dtype=np.float32), np.ones((S, HD), np.float32)))

    grid = (B, NM)
    upd_copy, ww_ret, pos, _maxv = pl.pallas_call(
        _main_body,
        grid=grid,
        in_specs=[
            pl.BlockSpec((1, TM, D), lambda b, i: (b, i, 0)),
            pl.BlockSpec((1, D, NS), lambda b, i: (b, 0, 0)),
            pl.BlockSpec((1, 1, NS), lambda b, i: (b, 0, 0)),
            pl.BlockSpec((1, NS, H), lambda b, i: (b, 0, 0)),
            pl.BlockSpec((NS, H), lambda b, i: (0, 0)),
            pl.BlockSpec((D, H), lambda b, i: (0, 0)),
            pl.BlockSpec((1, H), lambda b, i: (0, 0)),
            pl.BlockSpec((1, 1, 1), lambda b, i: (b, 0, 0)),
        ],
        out_specs=[
            pl.BlockSpec((1, TM, D), lambda b, i: (b, i, 0)),
            pl.BlockSpec((1, TM, H), lambda b, i: (b, i, 0)),
            pl.BlockSpec((1, 1, 1), lambda b, i: (b, 0, 0)),
            pl.BlockSpec((1, 1, 1), lambda b, i: (b, 0, 0)),
        ],
        out_shape=(
            jax.ShapeDtypeStruct((B, M, D), _F32),
            jax.ShapeDtypeStruct((B, M, H), _F32),
            jax.ShapeDtypeStruct((B, 1, 1), jnp.int32),
            jax.ShapeDtypeStruct((B, 1, 1), _F32),
        ),
    )(memory_bank, kk, kbias, vb, ones16, Wo, row2(bo), scale)

    vspec = pl.BlockSpec(memory_space=pltpu.VMEM)
    updated = pl.pallas_call(
        _scatter_body,
        in_specs=[
            pl.BlockSpec(memory_space=pl.ANY),
            pl.BlockSpec(memory_space=pl.ANY),
            pl.BlockSpec(memory_space=pltpu.SMEM),
            vspec, vspec, vspec, vspec, vspec, vspec, vspec, vspec, vspec,
            vspec, vspec, vspec, vspec,
        ],
        out_specs=pl.BlockSpec(memory_space=pl.ANY),
        out_shape=jax.ShapeDtypeStruct((B, M, D), _F32),
        input_output_aliases={0: 0},
        scratch_shapes=[pltpu.VMEM((B, D), _F32), pltpu.VMEM((B, D), _F32),
                        pltpu.SemaphoreType.DMA],
    )(upd_copy, memory_bank, pos.reshape(B), kk, kbias, vb, ones16, Wo,
      row2(bo), sel, selimp, exists, W_upd1, row2(b_upd1), W_upd2,
      row2(b_upd2))

    return updated, ww_ret, imp2.reshape(B, S)


# speedup vs baseline: 2.0629x; 1.1209x over previous
import functools

import jax
import jax.numpy as jnp
import numpy as np
from jax import lax
from jax.experimental import pallas as pl
from jax.experimental.pallas import tpu as pltpu

B, S, M, D, H, NH = 4, 32, 50000, 128, 128, 8
HD = H // NH
THR = 0.7
TM = 10000
NM = M // TM
NS = NH * S

_F32 = jnp.float32


def _prologue_body(ni_ref, w_in_ref, b_in_ref, wq_ref, bq_ref, wk_ref, bk_ref,
                   wv_ref, bv_ref, wi1_ref, bi1_ref, wi2_ref, bi2_ref, wo_ref,
                   imp_ref, kk_ref, kbias_ref, vb_ref, scale_ref, exists_ref,
                   sel_ref, selimp_ref):
    ni = ni_ref[...]
    ip = jnp.dot(ni, w_in_ref[...], preferred_element_type=_F32) + b_in_ref[...]
    h1 = jnp.maximum(jnp.dot(ip, wi1_ref[...], preferred_element_type=_F32)
                     + bi1_ref[...], 0.0)
    logit = jnp.dot(h1, wi2_ref[...], preferred_element_type=_F32) + bi2_ref[...]
    imp = jax.nn.sigmoid(logit)
    imp_ref[...] = imp
    kmat = jnp.dot(ip, wk_ref[...], preferred_element_type=_F32) + bk_ref[...]
    vmat = jnp.dot(ip, wv_ref[...], preferred_element_type=_F32) + bv_ref[...]
    wq_eff = jnp.dot(w_in_ref[...], wq_ref[...], preferred_element_type=_F32)
    bq_eff = jnp.dot(b_in_ref[...], wq_ref[...], preferred_element_type=_F32) \
        + bq_ref[...]

    t1 = (lax.broadcasted_iota(jnp.int32, (S, NS), 1) % S
          == lax.broadcasted_iota(jnp.int32, (S, NS), 0)).astype(_F32)
    hm = (lax.broadcasted_iota(jnp.int32, (H, NS), 0) // HD
          == lax.broadcasted_iota(jnp.int32, (H, NS), 1) // S).astype(_F32)
    t2 = (lax.broadcasted_iota(jnp.int32, (NS, S), 0) % S
          == lax.broadcasted_iota(jnp.int32, (NS, S), 1)).astype(_F32)
    m2 = (lax.broadcasted_iota(jnp.int32, (NS, H), 0) // S
          == lax.broadcasted_iota(jnp.int32, (NS, H), 1) // HD).astype(_F32)
    iota_s = lax.broadcasted_iota(jnp.int32, (S, 1), 0)
    inv_sqrt_hd = np.float32(1.0 / np.sqrt(HD))

    for b in range(B):
        k_b = kmat[b * S:(b + 1) * S, :]
        v_b = vmat[b * S:(b + 1) * S, :]
        kt_tiled = lax.dot_general(k_b, t1, (((0,), (0,)), ((), ())),
                                   preferred_element_type=_F32)
        kblock = kt_tiled * hm * inv_sqrt_hd
        kk_ref[b] = jnp.dot(wq_eff, kblock, preferred_element_type=_F32)
        kbias_ref[b] = jnp.dot(bq_eff, kblock, preferred_element_type=_F32)
        vb_ref[b] = jnp.dot(t2, v_b, preferred_element_type=_F32) * m2

        imp_b = imp[b * S:(b + 1) * S, :]
        scale_ref[b] = jnp.mean(imp_b).reshape(1, 1)
        mask_b = imp_b > THR
        exists_ref[b] = jnp.max(jnp.where(mask_b, 1.0, 0.0)).reshape(1, 1)
        last_b = jnp.maximum(jnp.max(jnp.where(mask_b, iota_s, -1)), 0)
        smask = iota_s == last_b
        ni_b = ni[b * S:(b + 1) * S, :]
        sel_ref[b] = jnp.sum(jnp.where(smask, ni_b, 0.0), axis=0, keepdims=True)
        selimp_ref[b] = jnp.sum(jnp.where(smask, imp_b, 0.0)).reshape(1, 1)


def _main_body(mb_ref, kk_ref, kbias_ref, vb_ref, ones_ref, wo_ref, bo_ref,
               scale_ref, upd_ref, wr_ref, pos_ref, maxv_ref):
    i = pl.program_id(1)
    mb = mb_ref[0]
    scores = jnp.dot(mb, kk_ref[0], preferred_element_type=_F32) + kbias_ref[0]
    e = jnp.exp(scores - jnp.max(scores, axis=1, keepdims=True))
    ctxu = jnp.dot(e, vb_ref[0], preferred_element_type=_F32)
    denom = jnp.dot(e, ones_ref[...], preferred_element_type=_F32)
    ctx = ctxu / denom
    ww = jnp.dot(ctx, wo_ref[...], preferred_element_type=_F32) \
        + bo_ref[...]
    wr_ref[0] = ww * scale_ref[0]
    upd_ref[0] = mb

    sm = jnp.mean(ww, axis=1, keepdims=True)
    tmax = jnp.max(sm)
    iota = lax.broadcasted_iota(jnp.int32, (TM, 1), 0)
    tidx = jnp.min(jnp.where(sm == tmax, iota, TM))
    tmax2 = tmax.reshape(1, 1)
    posv = (i * TM + tidx).reshape(1, 1)

    @pl.when(i == 0)
    def _():
        maxv_ref[0] = tmax2
        pos_ref[0] = posv

    @pl.when(i > 0)
    def _():
        better = tmax2 > maxv_ref[0]
        maxv_ref[0] = jnp.where(better, tmax2, maxv_ref[0])
        pos_ref[0] = jnp.where(better, posv, pos_ref[0])


def _scatter_body(upd_in_ref, mb_any_ref, pos_ref, kk_ref, kbias_ref, vb_ref,
                  ones_ref, wo_ref, bo_ref, sel_ref, selimp_ref, exists_ref,
                  wu1_ref, bu1_ref, wu2_ref, bu2_ref,
                  out_ref, old_scratch, new_scratch, sem):
    copies = []
    for b in range(B):
        c = pltpu.make_async_copy(mb_any_ref.at[b, pl.ds(pos_ref[b], 1)],
                                  old_scratch.at[pl.ds(b, 1)], sem)
        c.start()
        copies.append(c)
    for c in copies:
        c.wait()
    old = old_scratch[...]
    sel = sel_ref[:, 0, :]
    comb = jnp.concatenate([old, sel], axis=1)
    a1 = jnp.maximum(jnp.dot(comb, wu1_ref[...], preferred_element_type=_F32)
                     + bu1_ref[...], 0.0)
    upd = jnp.tanh(jnp.dot(a1, wu2_ref[...], preferred_element_type=_F32)
                   + bu2_ref[...])
    for b in range(B):
        scores = jnp.dot(old[b:b + 1], kk_ref[b],
                         preferred_element_type=_F32) + kbias_ref[b]
        e = jnp.exp(scores - jnp.max(scores, axis=1, keepdims=True))
        ctxu = jnp.dot(e, vb_ref[b], preferred_element_type=_F32)
        denom = jnp.dot(e, ones_ref[...], preferred_element_type=_F32)
        wwrow = jnp.dot(ctxu / denom, wo_ref[...],
                        preferred_element_type=_F32) + bo_ref[...]
        wwpos = wwrow * selimp_ref[b]
        newv = jnp.where(exists_ref[b] > 0.5,
                         old[b:b + 1] + upd[b:b + 1] * wwpos, old[b:b + 1])
        new_scratch[b:b + 1, :] = newv
    copies = []
    for b in range(B):
        c = pltpu.make_async_copy(new_scratch.at[pl.ds(b, 1)],
                                  out_ref.at[b, pl.ds(pos_ref[b], 1)], sem)
        c.start()
        copies.append(c)
    for c in copies:
        c.wait()


def kernel(new_info, memory_bank, W_in, b_in, Wq, bq, Wk, bk, Wv, bv, Wo, bo,
           W_imp1, b_imp1, W_imp2, b_imp2, W_upd1, b_upd1, W_upd2, b_upd2):
    ni2 = new_info.reshape(B * S, D)
    row2 = lambda x: x.reshape(1, -1)

    imp2, kk, kbias, vb, scale, exists, sel, selimp = pl.pallas_call(
        _prologue_body,
        out_shape=(
            jax.ShapeDtypeStruct((B * S, 1), _F32),
            jax.ShapeDtypeStruct((B, D, NS), _F32),
            jax.ShapeDtypeStruct((B, 1, NS), _F32),
            jax.ShapeDtypeStruct((B, NS, H), _F32),
            jax.ShapeDtypeStruct((B, 1, 1), _F32),
            jax.ShapeDtypeStruct((B, 1, 1), _F32),
            jax.ShapeDtypeStruct((B, 1, D), _F32),
            jax.ShapeDtypeStruct((B, 1, 1), _F32),
        ),
    )(ni2, W_in, row2(b_in), Wq, row2(bq), Wk, row2(bk), Wv, row2(bv),
      W_imp1, row2(b_imp1), W_imp2, row2(b_imp2), Wo)

    ones16 = jnp.asarray(
        np.kron(np.eye(NH, dtype=np.float32), np.ones((S, HD), np.float32)))

    grid = (B, NM)
    upd_copy, ww_ret, pos, _maxv = pl.pallas_call(
        _main_body,
        grid=grid,
        in_specs=[
            pl.BlockSpec((1, TM, D), lambda b, i: (b, i, 0)),
            pl.BlockSpec((1, D, NS), lambda b, i: (b, 0, 0)),
            pl.BlockSpec((1, 1, NS), lambda b, i: (b, 0, 0)),
            pl.BlockSpec((1, NS, H), lambda b, i: (b, 0, 0)),
            pl.BlockSpec((NS, H), lambda b, i: (0, 0)),
            pl.BlockSpec((D, H), lambda b, i: (0, 0)),
            pl.BlockSpec((1, H), lambda b, i: (0, 0)),
            pl.BlockSpec((1, 1, 1), lambda b, i: (b, 0, 0)),
        ],
        out_specs=[
            pl.BlockSpec((1, TM, D), lambda b, i: (b, i, 0)),
            pl.BlockSpec((1, TM, H), lambda b, i: (b, i, 0)),
            pl.BlockSpec((1, 1, 1), lambda b, i: (b, 0, 0)),
            pl.BlockSpec((1, 1, 1), lambda b, i: (b, 0, 0)),
        ],
        out_shape=(
            jax.ShapeDtypeStruct((B, M, D), _F32),
            jax.ShapeDtypeStruct((B, M, H), _F32),
            jax.ShapeDtypeStruct((B, 1, 1), jnp.int32),
            jax.ShapeDtypeStruct((B, 1, 1), _F32),
        ),
    )(memory_bank, kk, kbias, vb, ones16, Wo, row2(bo), scale)

    vspec = pl.BlockSpec(memory_space=pltpu.VMEM)
    updated = pl.pallas_call(
        _scatter_body,
        in_specs=[
            pl.BlockSpec(memory_space=pl.ANY),
            pl.BlockSpec(memory_space=pl.ANY),
            pl.BlockSpec(memory_space=pltpu.SMEM),
            vspec, vspec, vspec, vspec, vspec, vspec, vspec, vspec, vspec,
            vspec, vspec, vspec, vspec,
        ],
        out_specs=pl.BlockSpec(memory_space=pl.ANY),
        out_shape=jax.ShapeDtypeStruct((B, M, D), _F32),
        input_output_aliases={0: 0},
        scratch_shapes=[pltpu.VMEM((B, D), _F32), pltpu.VMEM((B, D), _F32),
                        pltpu.SemaphoreType.DMA],
    )(upd_copy, memory_bank, pos.reshape(B), kk, kbias, vb, ones16, Wo,
      row2(bo), sel, selimp, exists, W_upd1, row2(b_upd1), W_upd2,
      row2(b_upd2))

    return updated, ww_ret, imp2.reshape(B, S)
